# baseline (speedup 1.0000x reference)
import functools

import jax
import jax.numpy as jnp
from jax import lax
from jax.experimental import pallas as pl
from jax.experimental.pallas import tpu as pltpu
from jax.experimental.pallas import tpu_sc as plsc

RANK = 128
D_MODEL = 1024


def _sc_gather(table, idx3d, n_rows, g):
    info = plsc.get_sparse_core_info()
    nw = info.num_cores * info.num_subcores
    per_w = n_rows // nw
    n_g = per_w // g
    mesh = plsc.VectorSubcoreMesh(core_axis_name="c", subcore_axis_name="s")

    @functools.partial(
        pl.kernel,
        mesh=mesh,
        out_type=jax.ShapeDtypeStruct((n_rows, RANK), jnp.float32),
        scratch_types=[
            pltpu.VMEM((n_g, g), jnp.int32),
            pltpu.VMEM((g, RANK), jnp.float32),
            pltpu.SemaphoreType.DMA,
        ],
    )
    def k(table_hbm, idx_hbm, out_hbm, idx_v, rows_v, sem):
        wid = lax.axis_index("s") * info.num_cores + lax.axis_index("c")
        pltpu.sync_copy(idx_hbm.at[wid], idx_v)
        row_base = wid * per_w

        def body(j, carry):
            pltpu.async_copy(table_hbm.at[idx_v.at[j]], rows_v, sem).wait()
            pltpu.sync_copy(rows_v, out_hbm.at[pl.ds(row_base + j * g, g)])
            return carry

        lax.fori_loop(0, n_g, body, 0)

    return k(table, idx3d)


def _tc_project(emb, proj):
    m = emb.shape[0]
    bm = 2048

    def body(e_ref, p_ref, o_ref):
        o_ref[...] = jnp.dot(e_ref[...].astype(jnp.bfloat16), p_ref[...],
                             preferred_element_type=jnp.float32)

    return pl.pallas_call(
        body,
        grid=(m // bm,),
        in_specs=[
            pl.BlockSpec((bm, RANK), lambda i: (i, 0)),
            pl.BlockSpec((RANK, D_MODEL), lambda i: (0, 0)),
        ],
        out_specs=pl.BlockSpec((bm, D_MODEL), lambda i: (i, 0)),
        out_shape=jax.ShapeDtypeStruct((m, D_MODEL), jnp.float32),
    )(emb, proj)


def kernel(x, embed_low, project_up):
    b, l = x.shape
    n_chunks = 4
    g = 80
    bk = b // n_chunks
    rows_k = bk * l
    proj_bf16 = project_up.astype(jnp.bfloat16)
    n_g = rows_k // (32 * g)
    outs = []
    for k in range(n_chunks):
        idx3d = x[k * bk:(k + 1) * bk].reshape(32, n_g, g).astype(jnp.int32)
        emb = _sc_gather(embed_low, idx3d, rows_k, g)
        out2d = _tc_project(emb, proj_bf16)
        outs.append(out2d.reshape(bk, l, D_MODEL))
    return jnp.concatenate(outs, axis=0)


# speedup vs baseline: 2.3685x; 2.3685x over previous
import functools

import jax
import jax.numpy as jnp
from jax import lax
from jax.experimental import pallas as pl
from jax.experimental.pallas import tpu as pltpu
from jax.experimental.pallas import tpu_sc as plsc

RANK = 128
D_MODEL = 1024
G = 128


def _sc_gather(table, idx3d, n_rows):
    info = plsc.get_sparse_core_info()
    nw = info.num_cores * info.num_subcores
    per_w = n_rows // nw
    n_g = per_w // G
    mesh = plsc.VectorSubcoreMesh(core_axis_name="c", subcore_axis_name="s")

    @functools.partial(
        pl.kernel,
        mesh=mesh,
        out_type=jax.ShapeDtypeStruct((n_rows, RANK), jnp.float32),
        scratch_types=[
            pltpu.VMEM((n_g, G), jnp.int32),
            pltpu.VMEM((G, RANK), jnp.float32),
            pltpu.SemaphoreType.DMA,
        ],
    )
    def k(table_hbm, idx_hbm, out_hbm, idx_v, rows_v, sem):
        wid = lax.axis_index("s") * info.num_cores + lax.axis_index("c")
        pltpu.sync_copy(idx_hbm.at[wid], idx_v)
        row_base = wid * per_w

        def body(j, carry):
            pltpu.async_copy(table_hbm.at[idx_v.at[j]], rows_v, sem).wait()
            pltpu.sync_copy(rows_v, out_hbm.at[pl.ds(row_base + j * G, G)])
            return carry

        lax.fori_loop(0, n_g, body, 0)

    return k(table, idx3d)


def _tc_project(emb, proj, b, l):
    bb = 32
    la = l - 2
    n_steps = b // bb

    def body(e_ref, p_ref, o_hbm, om, ot, sem_m, sem_t):
        step = pl.program_id(0)
        slot = lax.rem(step, 2)

        def wait_slot(s):
            pltpu.make_async_copy(
                om.at[s], o_hbm.at[pl.ds(0, bb), pl.ds(0, la), :],
                sem_m.at[s]).wait()
            pltpu.make_async_copy(
                ot.at[s], o_hbm.at[pl.ds(0, bb), pl.ds(la, 2), :],
                sem_t.at[s]).wait()

        @pl.when(step >= 2)
        def _():
            wait_slot(slot)

        p = p_ref[...]
        e = e_ref[...].astype(jnp.bfloat16)
        for i in range(bb):
            r = jnp.dot(e[i * l:(i + 1) * l, :], p,
                        preferred_element_type=jnp.float32)
            om[slot, i] = r[:la]
            ot[slot, i] = r[la:]

        dst = o_hbm.at[pl.ds(step * bb, bb)]
        for i in range(bb):
            pltpu.make_async_copy(
                om.at[slot, pl.ds(i, 1)],
                o_hbm.at[pl.ds(step * bb + i, 1), pl.ds(0, la), :],
                sem_m.at[slot]).start()
        pltpu.make_async_copy(
            ot.at[slot], dst.at[:, pl.ds(la, 2), :], sem_t.at[slot]).start()

        @pl.when(step == n_steps - 1)
        def _():
            wait_slot(slot)
            wait_slot(1 - slot)

    return pl.pallas_call(
        body,
        grid=(n_steps,),
        in_specs=[
            pl.BlockSpec((bb * l, RANK), lambda i: (i, 0)),
            pl.BlockSpec((RANK, D_MODEL), lambda i: (0, 0)),
        ],
        out_specs=pl.BlockSpec(memory_space=pltpu.HBM),
        out_shape=jax.ShapeDtypeStruct((b, l, D_MODEL), jnp.float32),
        scratch_shapes=[
            pltpu.VMEM((2, bb, la, D_MODEL), jnp.float32),
            pltpu.VMEM((2, bb, 2, D_MODEL), jnp.float32),
            pltpu.SemaphoreType.DMA((2,)),
            pltpu.SemaphoreType.DMA((2,)),
        ],
    )(emb, proj)


def kernel(x, embed_low, project_up):
    b, l = x.shape
    n_rows = b * l
    idx3d = x.reshape(32, n_rows // (32 * G), G).astype(jnp.int32)
    emb = _sc_gather(embed_low, idx3d, n_rows)
    return _tc_project(emb, project_up.astype(jnp.bfloat16), b, l)
